# baseline (speedup 1.0000x reference)
import jax
import jax.numpy as jnp
from jax.experimental import pallas as pl


_ROWS_PER_BLOCK = 128


def _swap_body(x1_ref, x2_ref, o1_ref, o2_ref):
    parity = jax.lax.broadcasted_iota(jnp.int32, x1_ref.shape, 0) % 2
    mask = parity == 0
    a = x1_ref[...]
    b = x2_ref[...]
    o1_ref[...] = jnp.where(mask, b, a)
    o2_ref[...] = jnp.where(mask, a, b)


def kernel(x1, x2):
    N, c, h, w = x1.shape
    rows = N * c
    cols = h * w
    f1 = x1.reshape(rows, cols)
    f2 = x2.reshape(rows, cols)
    grid = (rows // _ROWS_PER_BLOCK,)
    spec = pl.BlockSpec((_ROWS_PER_BLOCK, cols), lambda i: (i, 0))
    o1, o2 = pl.pallas_call(
        _swap_body,
        grid=grid,
        in_specs=[spec, spec],
        out_specs=[spec, spec],
        out_shape=[
            jax.ShapeDtypeStruct((rows, cols), x1.dtype),
            jax.ShapeDtypeStruct((rows, cols), x2.dtype),
        ],
    )(f1, f2)
    return (o1.reshape(N, c, h, w), o2.reshape(N, c, h, w))


# speedup vs baseline: 1.1674x; 1.1674x over previous
import jax
import jax.numpy as jnp
from jax.experimental import pallas as pl


_C_BLOCK = 48


def _swap_body(x1_ref, x2_ref, o1_ref, o2_ref):
    parity = jax.lax.broadcasted_iota(jnp.int32, x1_ref.shape, 1) % 2
    mask = parity == 0
    a = x1_ref[...]
    b = x2_ref[...]
    o1_ref[...] = jnp.where(mask, b, a)
    o2_ref[...] = jnp.where(mask, a, b)


def kernel(x1, x2):
    N, c, h, w = x1.shape
    grid = (N, c // _C_BLOCK)
    spec = pl.BlockSpec((1, _C_BLOCK, h, w), lambda i, j: (i, j, 0, 0))
    o1, o2 = pl.pallas_call(
        _swap_body,
        grid=grid,
        in_specs=[spec, spec],
        out_specs=[spec, spec],
        out_shape=[
            jax.ShapeDtypeStruct((N, c, h, w), x1.dtype),
            jax.ShapeDtypeStruct((N, c, h, w), x2.dtype),
        ],
    )(x1, x2)
    return (o1, o2)


# speedup vs baseline: 1.1835x; 1.0138x over previous
import jax
import jax.numpy as jnp
from jax.experimental import pallas as pl


_C_BLOCK = 192


def _swap_body(x1_ref, x2_ref, o1_ref, o2_ref):
    parity = jax.lax.broadcasted_iota(jnp.int32, x1_ref.shape, 1) % 2
    mask = parity == 0
    a = x1_ref[...]
    b = x2_ref[...]
    o1_ref[...] = jnp.where(mask, b, a)
    o2_ref[...] = jnp.where(mask, a, b)


def kernel(x1, x2):
    N, c, h, w = x1.shape
    grid = (N, c // _C_BLOCK)
    spec = pl.BlockSpec((1, _C_BLOCK, h, w), lambda i, j: (i, j, 0, 0))
    o1, o2 = pl.pallas_call(
        _swap_body,
        grid=grid,
        in_specs=[spec, spec],
        out_specs=[spec, spec],
        out_shape=[
            jax.ShapeDtypeStruct((N, c, h, w), x1.dtype),
            jax.ShapeDtypeStruct((N, c, h, w), x2.dtype),
        ],
    )(x1, x2)
    return (o1, o2)
